# baseline (speedup 1.0000x reference)
import jax
import jax.numpy as jnp
from jax.experimental import pallas as pl
from jax.experimental.pallas import tpu as pltpu

_E = 320000
_R = _E // 128
_ONE_F32_BITS = 1065353216


def _merge_body(gw, gs, hs, gt, ht, out_w, out_s, out_t,
                gw_v, gs_v, hs_v, gt_v, ht_v, w_v, s_v, t_v, sem_in, sem_out):
    loads = [
        pltpu.make_async_copy(gw, gw_v, sem_in.at[0]),
        pltpu.make_async_copy(gs, gs_v, sem_in.at[1]),
        pltpu.make_async_copy(hs, hs_v, sem_in.at[2]),
        pltpu.make_async_copy(gt, gt_v, sem_in.at[3]),
        pltpu.make_async_copy(ht, ht_v, sem_in.at[4]),
    ]
    for h in loads:
        h.start()

    top = pl.ds(0, _R)
    bot = pl.ds(_R, _R)

    w_v[bot, :] = jnp.full((_R, 128), _ONE_F32_BITS, jnp.int32)
    loads[0].wait()
    w_v[top, :] = gw_v[...]
    store_w = pltpu.make_async_copy(w_v, out_w, sem_out.at[0])
    store_w.start()

    loads[1].wait()
    s_v[top, :] = gs_v[...]
    loads[2].wait()
    s_v[bot, :] = hs_v[...]
    store_s = pltpu.make_async_copy(s_v, out_s, sem_out.at[1])
    store_s.start()

    loads[3].wait()
    t_v[top, :] = gt_v[...]
    loads[4].wait()
    t_v[bot, :] = ht_v[...]
    store_t = pltpu.make_async_copy(t_v, out_t, sem_out.at[2])
    store_t.start()

    store_w.wait()
    store_s.wait()
    store_t.wait()


def kernel(gen_sources, gen_targets, gen_weights, given_sources, given_targets, node_embeddings):
    hbm = pl.BlockSpec(memory_space=pltpu.MemorySpace.HBM)
    ins = [
        jax.lax.bitcast_convert_type(gen_weights, jnp.int32).reshape(_R, 128),
        gen_sources.reshape(_R, 128),
        given_sources.reshape(_R, 128),
        gen_targets.reshape(_R, 128),
        given_targets.reshape(_R, 128),
    ]
    out_w, out_s, out_t = pl.pallas_call(
        _merge_body,
        in_specs=[hbm] * 5,
        out_specs=[hbm] * 3,
        out_shape=(
            jax.ShapeDtypeStruct((2 * _R, 128), jnp.int32),
            jax.ShapeDtypeStruct((2 * _R, 128), jnp.int32),
            jax.ShapeDtypeStruct((2 * _R, 128), jnp.int32),
        ),
        scratch_shapes=[pltpu.VMEM((_R, 128), jnp.int32)] * 5
        + [pltpu.VMEM((2 * _R, 128), jnp.int32)] * 3
        + [
            pltpu.SemaphoreType.DMA((5,)),
            pltpu.SemaphoreType.DMA((3,)),
        ],
    )(*ins)
    return (
        out_s.reshape(2 * _E),
        out_t.reshape(2 * _E),
        jax.lax.bitcast_convert_type(out_w.reshape(2 * _E), jnp.float32),
        node_embeddings,
    )


# speedup vs baseline: 1.6224x; 1.6224x over previous
import jax
import jax.numpy as jnp
from jax.experimental import pallas as pl
from jax.experimental.pallas import tpu as pltpu

_E = 320000


def _merge_body(gs, gt, gw, hs, ht, out_s, out_t, out_w,
                s_v, t_v, w_v, hs_v, ht_v, sem_in, sem_out):
    top = pl.ds(0, _E)
    bot = pl.ds(_E, _E)
    loads = [
        pltpu.make_async_copy(hs, hs_v, sem_in.at[0]),
        pltpu.make_async_copy(ht, ht_v, sem_in.at[1]),
        pltpu.make_async_copy(gw, w_v.at[top], sem_in.at[2]),
        pltpu.make_async_copy(gs, s_v.at[top], sem_in.at[3]),
        pltpu.make_async_copy(gt, t_v.at[top], sem_in.at[4]),
    ]
    for h in loads:
        h.start()

    w_v[bot] = jnp.ones((_E,), jnp.float32)
    loads[2].wait()
    store_w = pltpu.make_async_copy(w_v, out_w, sem_out.at[0])
    store_w.start()

    loads[0].wait()
    s_v[bot] = hs_v[...]
    loads[3].wait()
    store_s = pltpu.make_async_copy(s_v, out_s, sem_out.at[1])
    store_s.start()

    loads[1].wait()
    t_v[bot] = ht_v[...]
    loads[4].wait()
    store_t = pltpu.make_async_copy(t_v, out_t, sem_out.at[2])
    store_t.start()

    store_w.wait()
    store_s.wait()
    store_t.wait()


def kernel(gen_sources, gen_targets, gen_weights, given_sources, given_targets, node_embeddings):
    hbm = pl.BlockSpec(memory_space=pltpu.MemorySpace.HBM)
    out_s, out_t, out_w = pl.pallas_call(
        _merge_body,
        in_specs=[hbm] * 5,
        out_specs=[hbm] * 3,
        out_shape=(
            jax.ShapeDtypeStruct((2 * _E,), jnp.int32),
            jax.ShapeDtypeStruct((2 * _E,), jnp.int32),
            jax.ShapeDtypeStruct((2 * _E,), jnp.float32),
        ),
        scratch_shapes=[
            pltpu.VMEM((2 * _E,), jnp.int32),
            pltpu.VMEM((2 * _E,), jnp.int32),
            pltpu.VMEM((2 * _E,), jnp.float32),
            pltpu.VMEM((_E,), jnp.int32),
            pltpu.VMEM((_E,), jnp.int32),
            pltpu.SemaphoreType.DMA((5,)),
            pltpu.SemaphoreType.DMA((3,)),
        ],
    )(gen_sources, gen_targets, gen_weights, given_sources, given_targets)
    return out_s, out_t, out_w, node_embeddings
